# Initial kernel scaffold; baseline (speedup 1.0000x reference)
import jax
import jax.numpy as jnp
from jax.experimental import pallas as pl


def kernel(x, expert_indices, W, b):
    raise NotImplementedError("write your pallas kernel here")


# speedup vs baseline: 2.2358x; 2.2358x over previous
import functools

import jax
import jax.numpy as jnp
from jax import lax
from jax.experimental import pallas as pl
from jax.experimental.pallas import tpu as pltpu
from jax.experimental.pallas import tpu_sc as plsc

_B, _T, _D = 4, 2048, 2048
_E, _C = 8, 512
_OUT = 16384
_O_E = _OUT // _E
_N_ROWS = _B * _E * _C

_NC, _NS = 2, 16
_NW = _NC * _NS
_ROWS_PER_W = _N_ROWS // _NW
_CH = 32


def _make_sc_gather():
    mesh = plsc.VectorSubcoreMesh(core_axis_name="c", subcore_axis_name="s")

    @functools.partial(
        pl.kernel,
        mesh=mesh,
        out_type=jax.ShapeDtypeStruct((_N_ROWS, _D), jnp.float32),
        scratch_types=[
            pltpu.VMEM((_ROWS_PER_W,), jnp.int32),
            pltpu.VMEM((_CH, _D), jnp.float32),
            pltpu.SemaphoreType.DMA,
        ],
    )
    def gather(table_hbm, idx_hbm, out_hbm, idx_v, rows_v, sem):
        wid = lax.axis_index("s") * _NC + lax.axis_index("c")
        base = wid * _ROWS_PER_W
        pltpu.sync_copy(idx_hbm.at[pl.ds(base, _ROWS_PER_W)], idx_v)

        def step(i, carry):
            off = i * _CH
            pltpu.async_copy(
                table_hbm.at[idx_v.at[pl.ds(off, _CH)]], rows_v, sem
            ).wait()
            pltpu.sync_copy(rows_v, out_hbm.at[pl.ds(base + off, _CH)])
            return carry

        lax.fori_loop(0, _ROWS_PER_W // _CH, step, 0)

    return gather


_sc_gather = _make_sc_gather()


def _mm_body(sel_ref, w_ref, bias_ref, out_ref):
    acc = lax.dot_general(
        sel_ref[0, 0],
        w_ref[0],
        (((1,), (1,)), ((), ())),
        preferred_element_type=jnp.float32,
    )
    out_ref[0, 0] = acc + bias_ref[0]


def _expert_matmul(sel4, We, be):
    return pl.pallas_call(
        _mm_body,
        grid=(_E, _B),
        in_specs=[
            pl.BlockSpec((1, 1, _C, _D), lambda e, b: (b, e, 0, 0)),
            pl.BlockSpec((1, _O_E, _D), lambda e, b: (e, 0, 0)),
            pl.BlockSpec((1, 1, _O_E), lambda e, b: (e, 0, 0)),
        ],
        out_specs=pl.BlockSpec((1, 1, _C, _O_E), lambda e, b: (b, e, 0, 0)),
        out_shape=jax.ShapeDtypeStruct((_B, _E, _C, _O_E), jnp.float32),
    )(sel4, We, be)


def kernel(x, expert_indices, W, b):
    table = x.reshape(_B * _T, _D)
    flat_idx = (
        expert_indices
        + (jnp.arange(_B, dtype=jnp.int32) * _T)[:, None, None]
    ).reshape(_N_ROWS)
    sel = _sc_gather(table, flat_idx)
    sel4 = sel.reshape(_B, _E, _C, _D)
    We = W.reshape(_E, _O_E, _D)
    be = b.reshape(_E, 1, _O_E)
    return _expert_matmul(sel4, We, be)
